# baseline (speedup 1.0000x reference)
import functools

import jax
import jax.numpy as jnp
from jax import lax
from jax.experimental import pallas as pl
from jax.experimental.pallas import tpu as pltpu
from jax.experimental.pallas import tpu_sc as plsc

BATCH = 4096
EMBED_DIM = 64


def _make_sc_gather(B, D):
    info = plsc.get_sparse_core_info()
    NC, NS = info.num_cores, info.num_subcores
    NW = NC * NS
    assert B % (8 * NW) == 0
    b_per_w = B // NW
    mesh = plsc.VectorSubcoreMesh(core_axis_name="c", subcore_axis_name="s")

    @functools.partial(
        pl.kernel,
        mesh=mesh,
        out_type=[
            jax.ShapeDtypeStruct((B, D), jnp.float32),
            jax.ShapeDtypeStruct((B, D), jnp.float32),
        ],
        scratch_types=[
            pltpu.VMEM((b_per_w,), jnp.int32),
            pltpu.VMEM((b_per_w,), jnp.int32),
            pltpu.SemaphoreType.DMA,
        ],
        compiler_params=pltpu.CompilerParams(needs_layout_passes=False),
    )
    def gather_k(uid_hbm, mid_hbm, ut_hbm, mt_hbm, uout_hbm, mout_hbm,
                 uidx_v, midx_v, sem):
        wid = lax.axis_index("s") * NC + lax.axis_index("c")
        base = wid * b_per_w
        pltpu.sync_copy(uid_hbm.at[pl.ds(base, b_per_w)], uidx_v)
        pltpu.sync_copy(mid_hbm.at[pl.ds(base, b_per_w)], midx_v)
        copies = []
        for idx_v, t_hbm, o_hbm in (
            (uidx_v, ut_hbm, uout_hbm),
            (midx_v, mt_hbm, mout_hbm),
        ):
            for c in range(b_per_w // 16):
                vec = idx_v[pl.ds(c * 16, 16)]
                for k in range(16):
                    s = vec[k]
                    copies.append(pltpu.async_copy(
                        t_hbm.at[s], o_hbm.at[base + c * 16 + k], sem))
        for cp in copies:
            cp.wait()

    return gather_k


def _mlp_body(u_ref, m_ref, w1a_ref, w1b_ref, b1_ref, w2_ref, b2_ref,
              w3_ref, b3_ref, o_ref):
    h1 = jnp.dot(u_ref[...], w1a_ref[...], preferred_element_type=jnp.float32)
    h1 += jnp.dot(m_ref[...], w1b_ref[...], preferred_element_type=jnp.float32)
    h1 = jnp.maximum(h1 + b1_ref[...], 0.0)
    h2 = jnp.dot(h1, w2_ref[...], preferred_element_type=jnp.float32)
    h2 = jnp.maximum(h2 + b2_ref[...], 0.0)
    o_ref[...] = (
        jnp.dot(h2, w3_ref[...], preferred_element_type=jnp.float32)
        + b3_ref[...]
    )


def _make_mlp(B, D, BB):
    grid = (B // BB,)
    const = lambda i: (0, 0)
    return pl.pallas_call(
        _mlp_body,
        grid=grid,
        in_specs=[
            pl.BlockSpec((BB, D), lambda i: (i, 0)),
            pl.BlockSpec((BB, D), lambda i: (i, 0)),
            pl.BlockSpec((D, 256), const),
            pl.BlockSpec((D, 256), const),
            pl.BlockSpec((1, 256), const),
            pl.BlockSpec((256, 64), const),
            pl.BlockSpec((1, 64), const),
            pl.BlockSpec((64, 1), const),
            pl.BlockSpec((1, 1), const),
        ],
        out_specs=pl.BlockSpec((BB, 1), lambda i: (i, 0)),
        out_shape=jax.ShapeDtypeStruct((B, 1), jnp.float32),
    )


@jax.jit
def kernel(user_id, movie_id, user_table, movie_table, W1, b1, W2, b2, W3, b3):
    B = user_id.shape[0]
    D = user_table.shape[1]
    gather_k = _make_sc_gather(B, D)
    u_emb, m_emb = gather_k(
        user_id.astype(jnp.int32), movie_id.astype(jnp.int32),
        user_table, movie_table)
    mlp = _make_mlp(B, D, 1024)
    return mlp(
        u_emb, m_emb,
        W1[:D], W1[D:],
        b1.reshape(1, 256),
        W2, b2.reshape(1, 64),
        W3, b3.reshape(1, 1),
    )


# speedup vs baseline: 2.1916x; 2.1916x over previous
import functools

import jax
import jax.numpy as jnp
from jax import lax
from jax.experimental import pallas as pl
from jax.experimental.pallas import tpu as pltpu
from jax.experimental.pallas import tpu_sc as plsc

BATCH = 4096
EMBED_DIM = 64


def _make_sc_gather(B, D):
    info = plsc.get_sparse_core_info()
    NC, NS = info.num_cores, info.num_subcores
    NW = NC * NS
    assert B % (8 * NW) == 0
    b_per_w = B // NW
    mesh = plsc.VectorSubcoreMesh(core_axis_name="c", subcore_axis_name="s")

    @functools.partial(
        pl.kernel,
        mesh=mesh,
        out_type=[
            jax.ShapeDtypeStruct((B, D), jnp.float32),
            jax.ShapeDtypeStruct((B, D), jnp.float32),
        ],
        scratch_types=[
            pltpu.VMEM((b_per_w,), jnp.int32),
            pltpu.VMEM((b_per_w,), jnp.int32),
            pltpu.VMEM((b_per_w, D), jnp.float32),
            pltpu.VMEM((b_per_w, D), jnp.float32),
            pltpu.SemaphoreType.DMA,
        ],
        compiler_params=pltpu.CompilerParams(needs_layout_passes=False),
    )
    def gather_k(uid_hbm, mid_hbm, ut_hbm, mt_hbm, uout_hbm, mout_hbm,
                 uidx_v, midx_v, urows_v, mrows_v, sem):
        wid = lax.axis_index("s") * NC + lax.axis_index("c")
        base = wid * b_per_w
        pltpu.sync_copy(uid_hbm.at[pl.ds(base, b_per_w)], uidx_v)
        pltpu.sync_copy(mid_hbm.at[pl.ds(base, b_per_w)], midx_v)
        copies = []
        for idx_v, t_hbm, rows_v in (
            (uidx_v, ut_hbm, urows_v),
            (midx_v, mt_hbm, mrows_v),
        ):
            for c in range(b_per_w // 16):
                vec = idx_v[pl.ds(c * 16, 16)]
                for k in range(16):
                    s = vec[k]
                    copies.append(pltpu.async_copy(
                        t_hbm.at[s], rows_v.at[c * 16 + k], sem))
        for cp in copies:
            cp.wait()
        pltpu.sync_copy(urows_v, uout_hbm.at[pl.ds(base, b_per_w)])
        pltpu.sync_copy(mrows_v, mout_hbm.at[pl.ds(base, b_per_w)])

    return gather_k


def _mlp_body(u_ref, m_ref, w1a_ref, w1b_ref, b1_ref, w2_ref, b2_ref,
              w3_ref, b3_ref, o_ref):
    h1 = jnp.dot(u_ref[...], w1a_ref[...], preferred_element_type=jnp.float32)
    h1 += jnp.dot(m_ref[...], w1b_ref[...], preferred_element_type=jnp.float32)
    h1 = jnp.maximum(h1 + b1_ref[...], 0.0)
    h2 = jnp.dot(h1, w2_ref[...], preferred_element_type=jnp.float32)
    h2 = jnp.maximum(h2 + b2_ref[...], 0.0)
    o_ref[...] = (
        jnp.dot(h2, w3_ref[...], preferred_element_type=jnp.float32)
        + b3_ref[...]
    )


def _make_mlp(B, D, BB):
    grid = (B // BB,)
    const = lambda i: (0, 0)
    return pl.pallas_call(
        _mlp_body,
        grid=grid,
        in_specs=[
            pl.BlockSpec((BB, D), lambda i: (i, 0)),
            pl.BlockSpec((BB, D), lambda i: (i, 0)),
            pl.BlockSpec((D, 256), const),
            pl.BlockSpec((D, 256), const),
            pl.BlockSpec((1, 256), const),
            pl.BlockSpec((256, 64), const),
            pl.BlockSpec((1, 64), const),
            pl.BlockSpec((64, 1), const),
            pl.BlockSpec((1, 1), const),
        ],
        out_specs=pl.BlockSpec((BB, 1), lambda i: (i, 0)),
        out_shape=jax.ShapeDtypeStruct((B, 1), jnp.float32),
    )


@jax.jit
def kernel(user_id, movie_id, user_table, movie_table, W1, b1, W2, b2, W3, b3):
    B = user_id.shape[0]
    D = user_table.shape[1]
    gather_k = _make_sc_gather(B, D)
    u_emb, m_emb = gather_k(
        user_id.astype(jnp.int32), movie_id.astype(jnp.int32),
        user_table, movie_table)
    mlp = _make_mlp(B, D, 1024)
    return mlp(
        u_emb, m_emb,
        W1[:D], W1[D:],
        b1.reshape(1, 256),
        W2, b2.reshape(1, 64),
        W3, b3.reshape(1, 1),
    )


# speedup vs baseline: 2.1917x; 1.0000x over previous
import functools

import jax
import jax.numpy as jnp
from jax import lax
from jax.experimental import pallas as pl
from jax.experimental.pallas import tpu as pltpu
from jax.experimental.pallas import tpu_sc as plsc

BATCH = 4096
EMBED_DIM = 64


def _make_sc_gather(B, D):
    info = plsc.get_sparse_core_info()
    NC, NS = info.num_cores, info.num_subcores
    NW = NC * NS
    assert B % (8 * NW) == 0
    b_per_w = B // NW
    mesh = plsc.VectorSubcoreMesh(core_axis_name="c", subcore_axis_name="s")

    @functools.partial(
        pl.kernel,
        mesh=mesh,
        out_type=[
            jax.ShapeDtypeStruct((B, D), jnp.float32),
            jax.ShapeDtypeStruct((B, D), jnp.float32),
        ],
        scratch_types=[
            pltpu.VMEM((b_per_w,), jnp.int32),
            pltpu.VMEM((b_per_w,), jnp.int32),
            pltpu.VMEM((b_per_w, D), jnp.float32),
            pltpu.VMEM((b_per_w, D), jnp.float32),
            pltpu.SemaphoreType.DMA,
        ],
    )
    def gather_k(uid_hbm, mid_hbm, ut_hbm, mt_hbm, uout_hbm, mout_hbm,
                 uidx_v, midx_v, urows_v, mrows_v, sem):
        wid = lax.axis_index("s") * NC + lax.axis_index("c")
        base = wid * b_per_w
        pltpu.sync_copy(uid_hbm.at[pl.ds(base, b_per_w)], uidx_v)
        pltpu.sync_copy(mid_hbm.at[pl.ds(base, b_per_w)], midx_v)
        copies = []
        for idx_v, t_hbm, rows_v in (
            (uidx_v, ut_hbm, urows_v),
            (midx_v, mt_hbm, mrows_v),
        ):
            for c in range(b_per_w // 16):
                vec = idx_v[pl.ds(c * 16, 16)]
                for k in range(16):
                    s = vec[k]
                    copies.append(pltpu.async_copy(
                        t_hbm.at[s], rows_v.at[c * 16 + k], sem))
        for cp in copies:
            cp.wait()
        pltpu.sync_copy(urows_v, uout_hbm.at[pl.ds(base, b_per_w)])
        pltpu.sync_copy(mrows_v, mout_hbm.at[pl.ds(base, b_per_w)])

    return gather_k


def _mlp_body(u_ref, m_ref, w1a_ref, w1b_ref, b1_ref, w2_ref, b2_ref,
              w3_ref, b3_ref, o_ref):
    h1 = jnp.dot(u_ref[...], w1a_ref[...], preferred_element_type=jnp.float32)
    h1 += jnp.dot(m_ref[...], w1b_ref[...], preferred_element_type=jnp.float32)
    h1 = jnp.maximum(h1 + b1_ref[...], 0.0)
    h2 = jnp.dot(h1, w2_ref[...], preferred_element_type=jnp.float32)
    h2 = jnp.maximum(h2 + b2_ref[...], 0.0)
    o_ref[...] = (
        jnp.dot(h2, w3_ref[...], preferred_element_type=jnp.float32)
        + b3_ref[...]
    )


def _make_mlp(B, D, BB):
    grid = (B // BB,)
    const = lambda i: (0, 0)
    return pl.pallas_call(
        _mlp_body,
        grid=grid,
        in_specs=[
            pl.BlockSpec((BB, D), lambda i: (i, 0)),
            pl.BlockSpec((BB, D), lambda i: (i, 0)),
            pl.BlockSpec((D, 256), const),
            pl.BlockSpec((D, 256), const),
            pl.BlockSpec((1, 256), const),
            pl.BlockSpec((256, 64), const),
            pl.BlockSpec((1, 64), const),
            pl.BlockSpec((64, 1), const),
            pl.BlockSpec((1, 1), const),
        ],
        out_specs=pl.BlockSpec((BB, 1), lambda i: (i, 0)),
        out_shape=jax.ShapeDtypeStruct((B, 1), jnp.float32),
    )


@jax.jit
def kernel(user_id, movie_id, user_table, movie_table, W1, b1, W2, b2, W3, b3):
    B = user_id.shape[0]
    D = user_table.shape[1]
    gather_k = _make_sc_gather(B, D)
    u_emb, m_emb = gather_k(
        user_id.astype(jnp.int32), movie_id.astype(jnp.int32),
        user_table, movie_table)
    mlp = _make_mlp(B, D, 1024)
    return mlp(
        u_emb, m_emb,
        W1[:D], W1[D:],
        b1.reshape(1, 256),
        W2, b2.reshape(1, 64),
        W3, b3.reshape(1, 1),
    )


# speedup vs baseline: 2.9538x; 1.3477x over previous
import functools

import jax
import jax.numpy as jnp
from jax import lax
from jax.experimental import pallas as pl
from jax.experimental.pallas import tpu as pltpu
from jax.experimental.pallas import tpu_sc as plsc

BATCH = 4096
EMBED_DIM = 64


def _make_sc_gather(B, V, D):
    info = plsc.get_sparse_core_info()
    NC, NS = info.num_cores, info.num_subcores
    NW = NC * NS
    rows_per_w = D // NW * 2
    mesh = plsc.VectorSubcoreMesh(core_axis_name="c", subcore_axis_name="s")

    @functools.partial(
        pl.kernel,
        mesh=mesh,
        out_type=[
            jax.ShapeDtypeStruct((D, B), jnp.float32),
            jax.ShapeDtypeStruct((D, B), jnp.float32),
        ],
        scratch_types=[
            pltpu.VMEM((B,), jnp.int32),
            pltpu.VMEM((B,), jnp.int32),
            pltpu.VMEM((V,), jnp.float32),
            pltpu.VMEM((B,), jnp.float32),
        ],
        compiler_params=pltpu.CompilerParams(needs_layout_passes=False),
    )
    def gather_k(uid_hbm, mid_hbm, ut_hbm, mt_hbm, uout_hbm, mout_hbm,
                 uidx_v, midx_v, row_v, orow_v):
        wid = lax.axis_index("s") * NC + lax.axis_index("c")
        pltpu.sync_copy(uid_hbm, uidx_v)
        pltpu.sync_copy(mid_hbm, midx_v)

        def gather_row(idx_v):
            def body(g, _):
                for j in range(8):
                    iv = idx_v[pl.ds(g * 128 + j * 16, 16)]
                    orow_v[pl.ds(g * 128 + j * 16, 16)] = (
                        plsc.load_gather(row_v, [iv]))
                return 0
            lax.fori_loop(0, B // 128, body, 0)

        for idx_v, t_hbm, o_hbm in (
            (uidx_v, ut_hbm, uout_hbm),
            (midx_v, mt_hbm, mout_hbm),
        ):
            for k in range(rows_per_w):
                r = wid + NW * k
                pltpu.sync_copy(t_hbm.at[r], row_v)
                gather_row(idx_v)
                pltpu.sync_copy(orow_v, o_hbm.at[r])

    return gather_k


def _mlp_t_body(u_ref, m_ref, w1a_ref, w1b_ref, b1_ref, w2_ref, b2_ref,
                w3_ref, b3_ref, o_ref):
    cdim = (((0,), (0,)), ((), ()))
    h1 = lax.dot_general(w1a_ref[...], u_ref[...], cdim,
                         preferred_element_type=jnp.float32)
    h1 += lax.dot_general(w1b_ref[...], m_ref[...], cdim,
                          preferred_element_type=jnp.float32)
    h1 = jnp.maximum(h1 + b1_ref[...], 0.0)
    h2 = lax.dot_general(w2_ref[...], h1, cdim,
                         preferred_element_type=jnp.float32)
    h2 = jnp.maximum(h2 + b2_ref[...], 0.0)
    o_ref[...] = (
        lax.dot_general(w3_ref[...], h2, cdim,
                        preferred_element_type=jnp.float32)
        + b3_ref[...]
    )


def _make_mlp(B, D, BB):
    grid = (B // BB,)
    const = lambda i: (0, 0)
    return pl.pallas_call(
        _mlp_t_body,
        grid=grid,
        in_specs=[
            pl.BlockSpec((D, BB), lambda i: (0, i)),
            pl.BlockSpec((D, BB), lambda i: (0, i)),
            pl.BlockSpec((D, 256), const),
            pl.BlockSpec((D, 256), const),
            pl.BlockSpec((256, 1), const),
            pl.BlockSpec((256, 64), const),
            pl.BlockSpec((64, 1), const),
            pl.BlockSpec((64, 1), const),
            pl.BlockSpec((1, 1), const),
        ],
        out_specs=pl.BlockSpec((1, BB), lambda i: (0, i)),
        out_shape=jax.ShapeDtypeStruct((1, B), jnp.float32),
    )


@jax.jit
def kernel(user_id, movie_id, user_table, movie_table, W1, b1, W2, b2, W3, b3):
    B = user_id.shape[0]
    V, D = user_table.shape
    gather_k = _make_sc_gather(B, V, D)
    u_t, m_t = gather_k(
        user_id.astype(jnp.int32), movie_id.astype(jnp.int32),
        user_table.T, movie_table.T)
    mlp = _make_mlp(B, D, 1024)
    out_t = mlp(
        u_t, m_t,
        W1[:D], W1[D:],
        b1.reshape(256, 1),
        W2, b2.reshape(64, 1),
        W3, b3.reshape(1, 1),
    )
    return out_t.reshape(B, 1)


# speedup vs baseline: 4.0166x; 1.3598x over previous
import functools

import jax
import jax.numpy as jnp
from jax import lax
from jax.experimental import pallas as pl
from jax.experimental.pallas import tpu as pltpu
from jax.experimental.pallas import tpu_sc as plsc

BATCH = 4096
EMBED_DIM = 64


def _make_sc_gather(B, V, D):
    info = plsc.get_sparse_core_info()
    NC, NS = info.num_cores, info.num_subcores
    NW = NC * NS
    rows_per_w = D // NW
    mesh = plsc.VectorSubcoreMesh(core_axis_name="c", subcore_axis_name="s")

    @functools.partial(
        pl.kernel,
        mesh=mesh,
        out_type=[
            jax.ShapeDtypeStruct((D, B), jnp.float32),
            jax.ShapeDtypeStruct((D, B), jnp.float32),
        ],
        scratch_types=[
            pltpu.VMEM((B,), jnp.int32),
            pltpu.VMEM((B,), jnp.int32),
            pltpu.VMEM((V,), jnp.float32),
            pltpu.VMEM((4, B), jnp.float32),
        ],
        compiler_params=pltpu.CompilerParams(needs_layout_passes=False),
    )
    def gather_k(uid_hbm, mid_hbm, ut_hbm, mt_hbm, uout_hbm, mout_hbm,
                 uidx_v, midx_v, row_v, orows_v):
        wid = lax.axis_index("s") * NC + lax.axis_index("c")
        pltpu.sync_copy(uid_hbm, uidx_v)
        pltpu.sync_copy(mid_hbm, midx_v)

        def gather_row(idx_v, slot):
            def body(g, _):
                for j in range(8):
                    iv = idx_v[pl.ds(g * 128 + j * 16, 16)]
                    orows_v[slot, pl.ds(g * 128 + j * 16, 16)] = (
                        plsc.load_gather(row_v, [iv]))
                return 0
            lax.fori_loop(0, B // 128, body, 0)

        tabs = ((uidx_v, ut_hbm, uout_hbm), (midx_v, mt_hbm, mout_hbm))
        for t, (idx_v, t_hbm, _) in enumerate(tabs):
            for k in range(rows_per_w):
                r = wid + NW * k
                pltpu.sync_copy(t_hbm.at[r], row_v)
                gather_row(idx_v, t * rows_per_w + k)
        for t, (_, _, o_hbm) in enumerate(tabs):
            for k in range(rows_per_w):
                r = wid + NW * k
                pltpu.sync_copy(orows_v.at[t * rows_per_w + k], o_hbm.at[r])

    return gather_k


def _mlp_t_body(u_ref, m_ref, w1a_ref, w1b_ref, b1_ref, w2_ref, b2_ref,
                w3_ref, b3_ref, o_ref):
    cdim = (((0,), (0,)), ((), ()))
    h1 = lax.dot_general(w1a_ref[...], u_ref[...], cdim,
                         preferred_element_type=jnp.float32)
    h1 += lax.dot_general(w1b_ref[...], m_ref[...], cdim,
                          preferred_element_type=jnp.float32)
    h1 = jnp.maximum(h1 + b1_ref[...], 0.0)
    h2 = lax.dot_general(w2_ref[...], h1, cdim,
                         preferred_element_type=jnp.float32)
    h2 = jnp.maximum(h2 + b2_ref[...], 0.0)
    o_ref[...] = (
        lax.dot_general(w3_ref[...], h2, cdim,
                        preferred_element_type=jnp.float32)
        + b3_ref[...]
    )


def _make_mlp(B, D, BB):
    grid = (B // BB,)
    const = lambda i: (0, 0)
    return pl.pallas_call(
        _mlp_t_body,
        grid=grid,
        in_specs=[
            pl.BlockSpec((D, BB), lambda i: (0, i)),
            pl.BlockSpec((D, BB), lambda i: (0, i)),
            pl.BlockSpec((D, 256), const),
            pl.BlockSpec((D, 256), const),
            pl.BlockSpec((256, 1), const),
            pl.BlockSpec((256, 64), const),
            pl.BlockSpec((64, 1), const),
            pl.BlockSpec((64, 1), const),
            pl.BlockSpec((1, 1), const),
        ],
        out_specs=pl.BlockSpec((1, BB), lambda i: (0, i)),
        out_shape=jax.ShapeDtypeStruct((1, B), jnp.float32),
    )


@jax.jit
def kernel(user_id, movie_id, user_table, movie_table, W1, b1, W2, b2, W3, b3):
    B = user_id.shape[0]
    V, D = user_table.shape
    gather_k = _make_sc_gather(B, V, D)
    u_t, m_t = gather_k(
        user_id.astype(jnp.int32), movie_id.astype(jnp.int32),
        user_table.T, movie_table.T)
    mlp = _make_mlp(B, D, 1024)
    out_t = mlp(
        u_t, m_t,
        W1[:D], W1[D:],
        b1.reshape(256, 1),
        W2, b2.reshape(64, 1),
        W3, b3.reshape(1, 1),
    )
    return out_t.reshape(B, 1)
